# baseline (speedup 1.0000x reference)
import functools

import jax
import jax.numpy as jnp
from jax import lax
from jax.experimental import pallas as pl
from jax.experimental.pallas import tpu as pltpu
from jax.experimental.pallas import tpu_sc as plsc

BATCH = 16384
IN_F = 512
OUT_F = 64

NC = 2
NS = 16
L = 16
NW = NC * NS
ROWS_W = BATCH // NW
CHUNK = 64
NCHUNK = ROWS_W // CHUNK
NQ = CHUNK // L


def _sc_gather(x, columns):
    mesh = plsc.VectorSubcoreMesh(core_axis_name="c", subcore_axis_name="s")

    @functools.partial(
        pl.kernel,
        mesh=mesh,
        out_type=jax.ShapeDtypeStruct((OUT_F, BATCH), jnp.float32),
        compiler_params=pltpu.CompilerParams(
            needs_layout_passes=False,
            skip_device_barrier=True,
        ),
        scratch_types=[
            pltpu.VMEM((OUT_F + L,), jnp.int32),
            pltpu.VMEM((CHUNK, IN_F), jnp.float32),
            pltpu.VMEM((CHUNK, IN_F), jnp.float32),
            pltpu.VMEM((OUT_F, ROWS_W), jnp.float32),
            pltpu.SemaphoreType.DMA,
            pltpu.SemaphoreType.DMA,
            pltpu.SemaphoreType.DMA,
        ],
    )
    def k(x_hbm, cols_hbm, out_hbm, cols_v, in0, in1, ob, is0, is1, osem):
        wid = lax.axis_index("s") * NC + lax.axis_index("c")
        base = wid * ROWS_W
        pltpu.sync_copy(cols_hbm, cols_v.at[pl.ds(0, OUT_F)])
        row_vecs = [lax.iota(jnp.int32, L) + q * L for q in range(NQ)]
        ins = [in0, in1]
        isem = [is0, is1]

        def start_load(ci, slot):
            row0 = base + (ci % NCHUNK) * CHUNK
            return pltpu.async_copy(
                x_hbm.at[pl.ds(row0, CHUNK)], ins[slot], isem[slot]
            )

        def wait_load(slot):
            pltpu.make_async_copy(
                x_hbm.at[pl.ds(base, CHUNK)], ins[slot], isem[slot]
            ).wait()

        def compute(ib, ci):
            @plsc.parallel_loop(0, OUT_F, unroll=16)
            def j_body(j):
                cvec = cols_v[pl.ds(j, L)]
                csplat = jnp.zeros((L,), jnp.int32) + cvec[0]
                for q in range(NQ):
                    vals = plsc.load_gather(ib, [row_vecs[q], csplat])
                    ob[j, pl.ds(ci * CHUNK + q * L, L)] = vals

        start_load(0, 0)

        def group_body(t, _):
            start_load(2 * t + 1, 1)
            wait_load(0)
            compute(ins[0], 2 * t)
            start_load(2 * t + 2, 0)
            wait_load(1)
            compute(ins[1], 2 * t + 1)
            return 0

        lax.fori_loop(0, NCHUNK // 2, group_body, 0)
        wait_load(0)
        pltpu.async_copy(
            ob, out_hbm.at[:, pl.ds(base, ROWS_W)], osem
        ).wait()

    return k(x, columns)


def kernel(x, columns):
    return _sc_gather(x, columns).T


# speedup vs baseline: 1.0200x; 1.0200x over previous
import functools

import jax
import jax.numpy as jnp
from jax import lax
from jax.experimental import pallas as pl
from jax.experimental.pallas import tpu as pltpu
from jax.experimental.pallas import tpu_sc as plsc

BATCH = 16384
IN_F = 512
OUT_F = 64

NC = 2
NS = 16
L = 16
NW = NC * NS
ROWS_W = BATCH // NW
CHUNK = 64
NCHUNK = ROWS_W // CHUNK
NQ = CHUNK // L


def _sc_gather(x, columns):
    mesh = plsc.VectorSubcoreMesh(core_axis_name="c", subcore_axis_name="s")

    @functools.partial(
        pl.kernel,
        mesh=mesh,
        out_type=jax.ShapeDtypeStruct((OUT_F, BATCH), jnp.float32),
        compiler_params=pltpu.CompilerParams(
            needs_layout_passes=False,
            skip_device_barrier=True,
        ),
        scratch_types=[
            pltpu.VMEM((OUT_F + L,), jnp.int32),
            pltpu.VMEM((CHUNK, IN_F), jnp.float32),
            pltpu.VMEM((CHUNK, IN_F), jnp.float32),
            pltpu.VMEM((OUT_F, ROWS_W), jnp.float32),
            pltpu.SemaphoreType.DMA,
            pltpu.SemaphoreType.DMA,
            pltpu.SemaphoreType.DMA,
        ],
    )
    def k(x_hbm, cols_hbm, out_hbm, cols_v, in0, in1, ob, is0, is1, osem):
        wid = lax.axis_index("s") * NC + lax.axis_index("c")
        base = wid * ROWS_W
        pltpu.sync_copy(cols_hbm, cols_v.at[pl.ds(0, OUT_F)])
        row_vecs = [lax.iota(jnp.int32, L) + q * L for q in range(NQ)]
        ins = [in0, in1]
        isem = [is0, is1]

        def start_load(ci, slot):
            row0 = base + (ci % NCHUNK) * CHUNK
            return pltpu.async_copy(
                x_hbm.at[pl.ds(row0, CHUNK)], ins[slot], isem[slot]
            )

        def wait_load(slot):
            pltpu.make_async_copy(
                x_hbm.at[pl.ds(base, CHUNK)], ins[slot], isem[slot]
            ).wait()

        def compute(ib, ci):
            @plsc.parallel_loop(0, OUT_F, unroll=8)
            def j_body(j):
                cvec = cols_v[pl.ds(j, L)]
                csplat = jnp.zeros((L,), jnp.int32) + cvec[0]
                for q in range(NQ):
                    vals = plsc.load_gather(ib, [row_vecs[q], csplat])
                    ob[j, pl.ds(ci * CHUNK + q * L, L)] = vals

        start_load(0, 0)

        def group_body(t, _):
            start_load(2 * t + 1, 1)
            wait_load(0)
            compute(ins[0], 2 * t)
            start_load(2 * t + 2, 0)
            wait_load(1)
            compute(ins[1], 2 * t + 1)
            return 0

        lax.fori_loop(0, NCHUNK // 2, group_body, 0)
        wait_load(0)
        pltpu.async_copy(
            ob, out_hbm.at[:, pl.ds(base, ROWS_W)], osem
        ).wait()

    return k(x, columns)


def kernel(x, columns):
    return _sc_gather(x, columns).T
